# baseline (speedup 1.0000x reference)
import functools

import jax
import jax.numpy as jnp
import numpy as np
from jax import lax
from jax.experimental import pallas as pl
from jax.experimental.pallas import tpu as pltpu
from jax.experimental.pallas import tpu_sc as plsc

LANES = 16


def _stage1_body(n_classes, proto_ref, x0_ref, w_ref, b_ref, idx_ref, wout_ref):
    p = proto_ref[0]
    P, C = p.shape
    x = jnp.concatenate([p, x0_ref[0]], axis=0)
    N = x.shape[0]
    n_cp = P // n_classes

    p2 = jnp.sum(p * p, axis=1, keepdims=True)
    x2 = jnp.sum(x * x, axis=1, keepdims=True)
    g = jnp.dot(p, x.T, preferred_element_type=jnp.float32)
    d2 = p2 + x2.T - 2.0 * g
    dist = jnp.sqrt(jnp.maximum(d2, 0.0)) * (1.0 / np.sqrt(C))

    m = jnp.min(dist, axis=0, keepdims=True)
    pio = lax.broadcasted_iota(jnp.int32, (P, N), 0)
    idx = jnp.min(jnp.where(dist == m, pio, P), axis=0, keepdims=True)

    e = jnp.exp(-dist)
    c0 = jnp.sum(e[:n_cp], axis=0, keepdims=True) * (1.0 / n_cp)
    c1 = jnp.sum(e[n_cp:], axis=0, keepdims=True) * (1.0 / n_cp)
    tot = c0 + c1 + 1e-6
    cw = jnp.where(idx < n_cp, c0, c1) / tot

    z = jnp.sum(x * w_ref[...], axis=1, keepdims=True).T + b_ref[0, 0]
    sw = 1.0 / (1.0 + jnp.exp(-z))

    mask = (pio == idx).astype(jnp.float32)
    all_c = jnp.sum(mask * cw, axis=1, keepdims=True) + 1e-6
    all_s = jnp.sum(mask * sw, axis=1, keepdims=True) + 1e-6
    g_c = jnp.sum(mask * all_c, axis=0, keepdims=True)
    g_s = jnp.sum(mask * all_s, axis=0, keepdims=True)
    w = 0.5 * cw / g_c + 0.5 * sw / g_s

    idx_ref[0] = idx
    wout_ref[0] = w


def _stage1(prototypes, x0, W_score, b_score):
    B, P, C = prototypes.shape
    N0 = x0.shape[1]
    N = P + N0
    w_row = W_score.reshape(1, C)
    b_2d = b_score.reshape(1, 1)
    return pl.pallas_call(
        functools.partial(_stage1_body, 2),
        grid=(B,),
        in_specs=[
            pl.BlockSpec((1, P, C), lambda b: (b, 0, 0)),
            pl.BlockSpec((1, N0, C), lambda b: (b, 0, 0)),
            pl.BlockSpec((1, C), lambda b: (0, 0)),
            pl.BlockSpec((1, 1), lambda b: (0, 0)),
        ],
        out_specs=[
            pl.BlockSpec((1, 1, N), lambda b: (b, 0, 0)),
            pl.BlockSpec((1, 1, N), lambda b: (b, 0, 0)),
        ],
        out_shape=[
            jax.ShapeDtypeStruct((B, 1, N), jnp.int32),
            jax.ShapeDtypeStruct((B, 1, N), jnp.float32),
        ],
    )(prototypes, x0, w_row, b_2d)


def _make_stage2(B, P, C, N0):
    N = P + N0
    COLH = C // 2
    CH = 64
    NCH = N // CH
    NPC = P // CH
    CGR = COLH // LANES
    mesh = plsc.VectorSubcoreMesh(core_axis_name="c", subcore_axis_name="s",
                                  num_cores=2, num_subcores=16)

    @functools.partial(
        pl.kernel, mesh=mesh,
        out_type=jax.ShapeDtypeStruct((B * P, C), jnp.float32),
        scratch_types=[
            pltpu.VMEM((CH, COLH), jnp.float32),
            pltpu.VMEM((CH, LANES), jnp.float32),
            pltpu.VMEM((CH, LANES), jnp.int32),
            pltpu.VMEM((P, COLH), jnp.float32),
        ],
        compiler_params=pltpu.CompilerParams(needs_layout_passes=False),
    )
    def stage2(proto_hbm, x0_hbm, idx_hbm, w_hbm, out_hbm, buf, wbuf, ibuf, acc):
        cid = lax.axis_index("c")
        sid = lax.axis_index("s")
        tile = sid * 2 + cid
        b = tile // 2
        col0 = (tile % 2) * COLH

        def zero_row(t, _):
            for j in range(CGR):
                acc[t, pl.ds(j * LANES, LANES)] = jnp.zeros((LANES,), jnp.float32)
            return ()
        lax.fori_loop(0, P, zero_row, ())

        cvec = [lax.iota(jnp.int32, LANES) + j * LANES for j in range(CGR)]

        def scatter_row(t, _):
            wv = wbuf[t, :]
            rv = ibuf[t, :]
            for j in range(CGR):
                v = buf[t, pl.ds(j * LANES, LANES)] * wv
                plsc.addupdate_scatter(acc, [rv, cvec[j]], v)
            return ()

        def process(src, tok0):
            pltpu.sync_copy(src, buf)
            pltpu.sync_copy(w_hbm.at[b, pl.ds(tok0, CH), :], wbuf)
            pltpu.sync_copy(idx_hbm.at[b, pl.ds(tok0, CH), :], ibuf)
            lax.fori_loop(0, CH, scatter_row, ())

        for k in range(NCH):
            if k < NPC:
                src = proto_hbm.at[b, pl.ds(k * CH, CH), pl.ds(col0, COLH)]
            else:
                src = x0_hbm.at[b, pl.ds((k - NPC) * CH, CH), pl.ds(col0, COLH)]
            process(src, k * CH)

        pltpu.sync_copy(acc, out_hbm.at[pl.ds(b * P, P), pl.ds(col0, COLH)])

    return stage2


def kernel(prototypes, x0, W_score, b_score):
    B, P, C = prototypes.shape
    N0 = x0.shape[1]
    N = P + N0
    idx3, w3 = _stage1(prototypes, x0, W_score, b_score)
    stage2 = _make_stage2(B, P, C, N0)
    idx_exp = jnp.broadcast_to(idx3.reshape(B, N, 1), (B, N, LANES))
    w_exp = jnp.broadcast_to(w3.reshape(B, N, 1), (B, N, LANES))
    out = stage2(prototypes, x0, idx_exp, w_exp)
    return (out.reshape(B, P, C), idx3.reshape(B, N))


# speedup vs baseline: 2.3252x; 2.3252x over previous
import functools

import jax
import jax.numpy as jnp
import numpy as np
from jax import lax
from jax.experimental import pallas as pl
from jax.experimental.pallas import tpu as pltpu
from jax.experimental.pallas import tpu_sc as plsc

LANES = 16


def _stage1_body(n_classes, proto_ref, x0_ref, w_ref, b_ref, idx_ref, wout_ref):
    p = proto_ref[0]
    P, C = p.shape
    x = jnp.concatenate([p, x0_ref[0]], axis=0)
    N = x.shape[0]
    n_cp = P // n_classes

    p2 = jnp.sum(p * p, axis=1, keepdims=True)
    x2 = jnp.sum(x * x, axis=1, keepdims=True)
    g = jnp.dot(p, x.T, preferred_element_type=jnp.float32)
    d2 = p2 + x2.T - 2.0 * g
    dist = jnp.sqrt(jnp.maximum(d2, 0.0)) * (1.0 / np.sqrt(C))

    m = jnp.min(dist, axis=0, keepdims=True)
    pio = lax.broadcasted_iota(jnp.int32, (P, N), 0)
    idx = jnp.min(jnp.where(dist == m, pio, P), axis=0, keepdims=True)

    e = jnp.exp(-dist)
    c0 = jnp.sum(e[:n_cp], axis=0, keepdims=True) * (1.0 / n_cp)
    c1 = jnp.sum(e[n_cp:], axis=0, keepdims=True) * (1.0 / n_cp)
    tot = c0 + c1 + 1e-6
    cw = jnp.where(idx < n_cp, c0, c1) / tot

    z = jnp.sum(x * w_ref[...], axis=1, keepdims=True).T + b_ref[0, 0]
    sw = 1.0 / (1.0 + jnp.exp(-z))

    mask = (pio == idx).astype(jnp.float32)
    all_c = jnp.sum(mask * cw, axis=1, keepdims=True) + 1e-6
    all_s = jnp.sum(mask * sw, axis=1, keepdims=True) + 1e-6
    g_c = jnp.sum(mask * all_c, axis=0, keepdims=True)
    g_s = jnp.sum(mask * all_s, axis=0, keepdims=True)
    w = 0.5 * cw / g_c + 0.5 * sw / g_s

    idx_ref[0] = idx
    wout_ref[0] = w


def _stage1(prototypes, x0, W_score, b_score):
    B, P, C = prototypes.shape
    N0 = x0.shape[1]
    N = P + N0
    w_row = W_score.reshape(1, C)
    b_2d = b_score.reshape(1, 1)
    return pl.pallas_call(
        functools.partial(_stage1_body, 2),
        grid=(B,),
        in_specs=[
            pl.BlockSpec((1, P, C), lambda b: (b, 0, 0)),
            pl.BlockSpec((1, N0, C), lambda b: (b, 0, 0)),
            pl.BlockSpec((1, C), lambda b: (0, 0)),
            pl.BlockSpec((1, 1), lambda b: (0, 0)),
        ],
        out_specs=[
            pl.BlockSpec((1, 1, N), lambda b: (b, 0, 0)),
            pl.BlockSpec((1, 1, N), lambda b: (b, 0, 0)),
        ],
        out_shape=[
            jax.ShapeDtypeStruct((B, 1, N), jnp.int32),
            jax.ShapeDtypeStruct((B, 1, N), jnp.float32),
        ],
    )(prototypes, x0, w_row, b_2d)


def _make_stage2(B, P, C, N0):
    N = P + N0
    COLH = C // 2
    CH = 64
    NCH = N // CH
    NPC = P // CH
    CGR = COLH // LANES
    mesh = plsc.VectorSubcoreMesh(core_axis_name="c", subcore_axis_name="s",
                                  num_cores=2, num_subcores=16)

    @functools.partial(
        pl.kernel, mesh=mesh,
        out_type=jax.ShapeDtypeStruct((B * P, C), jnp.float32),
        scratch_types=[
            pltpu.VMEM((2, CH, COLH), jnp.float32),
            pltpu.VMEM((2, CH, LANES), jnp.float32),
            pltpu.VMEM((2, CH, LANES), jnp.int32),
            pltpu.VMEM((P, COLH), jnp.float32),
            pltpu.SemaphoreType.DMA((2,)),
        ],
        compiler_params=pltpu.CompilerParams(needs_layout_passes=False),
    )
    def stage2(proto_hbm, x0_hbm, idx_hbm, w_hbm, out_hbm, buf, wbuf, ibuf, acc, sems):
        cid = lax.axis_index("c")
        sid = lax.axis_index("s")
        tile = sid * 2 + cid
        b = tile // 2
        col0 = (tile % 2) * COLH

        @functools.partial(plsc.parallel_loop, 0, P, unroll=4)
        def _(t):
            for j in range(CGR):
                acc[t, pl.ds(j * LANES, LANES)] = jnp.zeros((LANES,), jnp.float32)

        cvec = [lax.iota(jnp.int32, LANES) + j * LANES for j in range(CGR)]

        def issue(k, slot):
            if k < NPC:
                src = proto_hbm.at[b, pl.ds(k * CH, CH), pl.ds(col0, COLH)]
            else:
                src = x0_hbm.at[b, pl.ds((k - NPC) * CH, CH), pl.ds(col0, COLH)]
            return [
                pltpu.async_copy(src, buf.at[slot], sems.at[slot]),
                pltpu.async_copy(w_hbm.at[b, pl.ds(k * CH, CH), :],
                                 wbuf.at[slot], sems.at[slot]),
                pltpu.async_copy(idx_hbm.at[b, pl.ds(k * CH, CH), :],
                                 ibuf.at[slot], sems.at[slot]),
            ]

        pending = {0: issue(0, 0)}
        for k in range(NCH):
            slot = k % 2
            if k + 1 < NCH:
                pending[k + 1] = issue(k + 1, (k + 1) % 2)
            for d in pending.pop(k):
                d.wait()
            bslot, wslot, islot = buf.at[slot], wbuf.at[slot], ibuf.at[slot]

            @functools.partial(plsc.parallel_loop, 0, CH, unroll=2)
            def _(t):
                wv = wslot[t, :]
                rv = islot[t, :]
                for j in range(CGR):
                    v = bslot[t, pl.ds(j * LANES, LANES)] * wv
                    plsc.addupdate_scatter(acc, [rv, cvec[j]], v)

        pltpu.sync_copy(acc, out_hbm.at[pl.ds(b * P, P), pl.ds(col0, COLH)])

    return stage2


def kernel(prototypes, x0, W_score, b_score):
    B, P, C = prototypes.shape
    N0 = x0.shape[1]
    N = P + N0
    idx3, w3 = _stage1(prototypes, x0, W_score, b_score)
    stage2 = _make_stage2(B, P, C, N0)
    idx_exp = jnp.broadcast_to(idx3.reshape(B, N, 1), (B, N, LANES))
    w_exp = jnp.broadcast_to(w3.reshape(B, N, 1), (B, N, LANES))
    out = stage2(prototypes, x0, idx_exp, w_exp)
    return (out.reshape(B, P, C), idx3.reshape(B, N))
